# baseline (speedup 1.0000x reference)
import functools

import jax
import jax.numpy as jnp
from jax import lax
from jax.experimental import pallas as pl
from jax.experimental.pallas import tpu as pltpu

_LANE = 128
_TM = 1024
_VMEM = 64 * 1024 * 1024


def _ceil_to(v, m):
    return ((v + m - 1) // m) * m


_QSCALE = 128.0


def _feat_kernel(x_ref, w_ref, o_ref):
    o_ref[...] = jnp.dot(
        x_ref[...].astype(jnp.bfloat16), w_ref[...],
        preferred_element_type=jnp.float32,
    ).astype(o_ref.dtype)


def _agg1_kernel(adj_ref, sup_ref, b1_ref, w23_ref, o_ref, adjq_ref):
    a8 = (adj_ref[...] * _QSCALE).astype(adjq_ref.dtype)
    adjq_ref[...] = a8
    acc = jnp.dot(a8, sup_ref[...], preferred_element_type=jnp.float32)
    h = acc * (1.0 / _QSCALE) + b1_ref[...]
    h = jnp.maximum(h, 0.0).astype(jnp.bfloat16)
    o_ref[...] = jnp.dot(
        h, w23_ref[...], preferred_element_type=jnp.float32
    ).astype(o_ref.dtype)


def _agg2_kernel(adjq_ref, sup_ref, b23_ref, o1_ref, o2_ref, o3_ref, *,
                 c1, c2):
    z = jnp.dot(adjq_ref[...], sup_ref[...], preferred_element_type=jnp.float32)
    z = z * (1.0 / _QSCALE) + b23_ref[...]

    lane = lax.broadcasted_iota(jnp.int32, z.shape, 1)
    minus_inf = jnp.float32(-jnp.inf)

    def group_stats(mask):
        zg = jnp.where(mask, z, minus_inf)
        m = jnp.max(zg, axis=-1, keepdims=True)
        e = jnp.exp(zg - m)
        return m, e, jnp.sum(e, axis=-1, keepdims=True)

    m1, e1, s1 = group_stats(lane < c1)
    m2, _, s2 = group_stats((lane >= c1) & (lane < c1 + c2))
    o1_ref[...] = (z - m1 - jnp.log(s1))[:, :c1]
    o2_ref[...] = (z - m2 - jnp.log(s2))[:, c1:c1 + c2]
    o3_ref[...] = e1[:, c1 - 1:c1] / s1


def _row_slab_call(body, n_p, out_shapes, out_specs, operands, operand_specs):
    return pl.pallas_call(
        body,
        out_shape=out_shapes,
        grid=(n_p // _TM,),
        in_specs=operand_specs,
        out_specs=out_specs,
        compiler_params=pltpu.CompilerParams(
            dimension_semantics=("parallel",),
            vmem_limit_bytes=_VMEM,
        ),
    )(*operands)


def kernel(gc1_w, gc1_b, gc2_w, gc2_b, gc3_w, gc3_b, x, adj):
    n, nfeat = x.shape
    nhid = gc1_w.shape[1]
    c1 = gc2_w.shape[1]
    c2 = gc3_w.shape[1]
    f23 = _ceil_to(c1 + c2 + 1, _LANE)

    n_p = _ceil_to(n, _TM)
    if n_p != n:
        x = jnp.pad(x, ((0, n_p - n), (0, 0)))
        adj = jnp.pad(adj, ((0, n_p - n), (0, n_p - n)))

    w1 = gc1_w.astype(jnp.bfloat16)
    b1 = gc1_b.reshape(1, nhid)
    w23 = jnp.pad(
        jnp.concatenate([gc2_w, gc3_w], axis=1),
        ((0, 0), (0, f23 - c1 - c2)),
    ).astype(jnp.bfloat16)
    b23 = jnp.pad(
        jnp.concatenate([gc2_b, gc3_b]), (0, f23 - c1 - c2)
    ).reshape(1, f23)

    whole = lambda shape: pl.BlockSpec(shape, lambda i: (0,) * len(shape))
    row_slab = lambda cols: pl.BlockSpec((_TM, cols), lambda i: (i, 0))
    f8 = jnp.float8_e4m3fn

    sup1 = _row_slab_call(
        _feat_kernel, n_p,
        jax.ShapeDtypeStruct((n_p, nhid), f8), row_slab(nhid),
        (x, w1),
        [row_slab(nfeat), whole((nfeat, nhid))],
    )
    sup23, adjq = _row_slab_call(
        _agg1_kernel, n_p,
        (jax.ShapeDtypeStruct((n_p, f23), f8),
         jax.ShapeDtypeStruct((n_p, n_p), f8)),
        (row_slab(f23), row_slab(n_p)),
        (adj, sup1, b1, w23),
        [row_slab(n_p), whole((n_p, nhid)), whole((1, nhid)),
         whole((nhid, f23))],
    )
    ls1, ls2, p_last = _row_slab_call(
        functools.partial(_agg2_kernel, c1=c1, c2=c2),
        n_p,
        (jax.ShapeDtypeStruct((n_p, c1), jnp.float32),
         jax.ShapeDtypeStruct((n_p, c2), jnp.float32),
         jax.ShapeDtypeStruct((n_p, 1), jnp.float32)),
        (row_slab(c1), row_slab(c2), row_slab(1)),
        (adjq, sup23, b23),
        [row_slab(n_p), whole((n_p, f23)), whole((1, f23))],
    )

    return ls1[:n], ls2[:n], p_last[:n, 0]


# speedup vs baseline: 1.0839x; 1.0839x over previous
import functools

import jax
import jax.numpy as jnp
from jax import lax
from jax.experimental import pallas as pl
from jax.experimental.pallas import tpu as pltpu

_LANE = 128
_TM = 1024
_VMEM = 64 * 1024 * 1024
_QSCALE = 128.0


def _ceil_to(v, m):
    return ((v + m - 1) // m) * m


def _agg1_kernel(adj_ref, x_ref, w1_ref, b1_ref, w23_ref, o_ref, adjq_ref):
    a = adj_ref[...]
    adjq_ref[...] = (a * _QSCALE).astype(adjq_ref.dtype)
    t = jnp.dot(
        a.astype(jnp.bfloat16), x_ref[...].astype(jnp.bfloat16),
        preferred_element_type=jnp.float32,
    )
    h = jnp.dot(
        t.astype(jnp.bfloat16), w1_ref[...],
        preferred_element_type=jnp.float32,
    )
    h = jnp.maximum(h + b1_ref[...], 0.0).astype(jnp.bfloat16)
    o_ref[...] = jnp.dot(
        h, w23_ref[...], preferred_element_type=jnp.float32
    ).astype(o_ref.dtype)


def _agg2_kernel(adjq_ref, sup_ref, b23_ref, o_ref, *, c1, c2):
    z = jnp.dot(adjq_ref[...], sup_ref[...], preferred_element_type=jnp.float32)
    z = z * (1.0 / _QSCALE) + b23_ref[...]

    lane = lax.broadcasted_iota(jnp.int32, z.shape, 1)
    minus_inf = jnp.float32(-jnp.inf)

    def group_stats(mask):
        zg = jnp.where(mask, z, minus_inf)
        m = jnp.max(zg, axis=-1, keepdims=True)
        e = jnp.exp(zg - m)
        return m, e, jnp.sum(e, axis=-1, keepdims=True)

    mask1 = lane < c1
    mask2 = (lane >= c1) & (lane < c1 + c2)
    m1, e1, s1 = group_stats(mask1)
    m2, _, s2 = group_stats(mask2)
    prob_last = (
        jnp.sum(jnp.where(lane == c1 - 1, e1, 0.0), axis=-1, keepdims=True)
        / s1
    )
    out = jnp.where(
        mask1,
        z - m1 - jnp.log(s1),
        jnp.where(mask2, z - m2 - jnp.log(s2), prob_last),
    )
    o_ref[...] = out


def _row_slab_call(body, n_p, out_shapes, out_specs, operands, operand_specs):
    return pl.pallas_call(
        body,
        out_shape=out_shapes,
        grid=(n_p // _TM,),
        in_specs=operand_specs,
        out_specs=out_specs,
        compiler_params=pltpu.CompilerParams(
            dimension_semantics=("parallel",),
            vmem_limit_bytes=_VMEM,
        ),
    )(*operands)


def kernel(gc1_w, gc1_b, gc2_w, gc2_b, gc3_w, gc3_b, x, adj):
    n, nfeat = x.shape
    nhid = gc1_w.shape[1]
    c1 = gc2_w.shape[1]
    c2 = gc3_w.shape[1]
    f23 = _ceil_to(c1 + c2 + 1, _LANE)

    n_p = _ceil_to(n, _TM)
    if n_p != n:
        x = jnp.pad(x, ((0, n_p - n), (0, 0)))
        adj = jnp.pad(adj, ((0, n_p - n), (0, n_p - n)))

    w1 = gc1_w.astype(jnp.bfloat16)
    b1 = gc1_b.reshape(1, nhid)
    w23 = jnp.pad(
        jnp.concatenate([gc2_w, gc3_w], axis=1),
        ((0, 0), (0, f23 - c1 - c2)),
    ).astype(jnp.bfloat16)
    b23 = jnp.pad(
        jnp.concatenate([gc2_b, gc3_b]), (0, f23 - c1 - c2)
    ).reshape(1, f23)

    whole = lambda shape: pl.BlockSpec(shape, lambda i: (0,) * len(shape))
    row_slab = lambda cols: pl.BlockSpec((_TM, cols), lambda i: (i, 0))
    f8 = jnp.float8_e4m3fn

    sup23, adjq = _row_slab_call(
        _agg1_kernel, n_p,
        (jax.ShapeDtypeStruct((n_p, f23), f8),
         jax.ShapeDtypeStruct((n_p, n_p), f8)),
        (row_slab(f23), row_slab(n_p)),
        (adj, x, w1, b1, w23),
        [row_slab(n_p), whole((n_p, nfeat)), whole((nfeat, nhid)),
         whole((1, nhid)), whole((nhid, f23))],
    )
    slab = _row_slab_call(
        functools.partial(_agg2_kernel, c1=c1, c2=c2),
        n_p,
        jax.ShapeDtypeStruct((n_p, f23), jnp.float32), row_slab(f23),
        (adjq, sup23, b23),
        [row_slab(n_p), whole((n_p, f23)), whole((1, f23))],
    )

    return slab[:n, :c1], slab[:n, c1:c1 + c2], slab[:n, c1 + c2]


# speedup vs baseline: 1.1059x; 1.0203x over previous
import functools

import jax
import jax.numpy as jnp
from jax import lax
from jax.experimental import pallas as pl
from jax.experimental.pallas import tpu as pltpu

_LANE = 128
_TM1 = 1024
_TM2 = 2048
_VMEM = 64 * 1024 * 1024
_QSCALE = 128.0


def _ceil_to(v, m):
    return ((v + m - 1) // m) * m


def _agg1_kernel(adj_ref, x_ref, w1_ref, b1_ref, w23_ref, o_ref, adjq_ref):
    a = adj_ref[...]
    adjq_ref[...] = (a * _QSCALE).astype(adjq_ref.dtype)
    t = jnp.dot(
        a.astype(jnp.bfloat16), x_ref[...].astype(jnp.bfloat16),
        preferred_element_type=jnp.float32,
    )
    h = jnp.dot(t, w1_ref[...], preferred_element_type=jnp.float32)
    h = jnp.maximum(h + b1_ref[...], 0.0).astype(jnp.bfloat16)
    o_ref[...] = jnp.dot(
        h, w23_ref[...], preferred_element_type=jnp.float32
    ).astype(o_ref.dtype)


def _agg2_kernel(adjq_ref, sup_ref, b23_ref, o_ref, *, c1, c2):
    z = jnp.dot(adjq_ref[...], sup_ref[...], preferred_element_type=jnp.float32)
    z = z * (1.0 / _QSCALE) + b23_ref[...]

    lane = lax.broadcasted_iota(jnp.int32, z.shape, 1)
    minus_inf = jnp.float32(-jnp.inf)

    def group_stats(mask):
        zg = jnp.where(mask, z, minus_inf)
        m = jnp.max(zg, axis=-1, keepdims=True)
        e = jnp.exp(zg - m)
        return m, e, jnp.sum(e, axis=-1, keepdims=True)

    mask1 = lane < c1
    mask2 = (lane >= c1) & (lane < c1 + c2)
    m1, e1, s1 = group_stats(mask1)
    m2, _, s2 = group_stats(mask2)
    prob_last = (
        jnp.sum(jnp.where(lane == c1 - 1, e1, 0.0), axis=-1, keepdims=True)
        / s1
    )
    out = jnp.where(
        mask1,
        z - m1 - jnp.log(s1),
        jnp.where(mask2, z - m2 - jnp.log(s2), prob_last),
    )
    o_ref[...] = out


def _row_slab_call(body, n_p, tm, out_shapes, out_specs, operands,
                   operand_specs):
    return pl.pallas_call(
        body,
        out_shape=out_shapes,
        grid=(n_p // tm,),
        in_specs=operand_specs,
        out_specs=out_specs,
        compiler_params=pltpu.CompilerParams(
            dimension_semantics=("parallel",),
            vmem_limit_bytes=_VMEM,
        ),
    )(*operands)


def kernel(gc1_w, gc1_b, gc2_w, gc2_b, gc3_w, gc3_b, x, adj):
    n, nfeat = x.shape
    nhid = gc1_w.shape[1]
    c1 = gc2_w.shape[1]
    c2 = gc3_w.shape[1]
    f23 = _ceil_to(c1 + c2 + 1, _LANE)

    n_p = _ceil_to(n, max(_TM1, _TM2))
    if n_p != n:
        x = jnp.pad(x, ((0, n_p - n), (0, 0)))
        adj = jnp.pad(adj, ((0, n_p - n), (0, n_p - n)))

    w1 = gc1_w
    b1 = gc1_b.reshape(1, nhid)
    w23 = jnp.pad(
        jnp.concatenate([gc2_w, gc3_w], axis=1),
        ((0, 0), (0, f23 - c1 - c2)),
    ).astype(jnp.bfloat16)
    b23 = jnp.pad(
        jnp.concatenate([gc2_b, gc3_b]), (0, f23 - c1 - c2)
    ).reshape(1, f23)

    whole = lambda shape: pl.BlockSpec(shape, lambda i: (0,) * len(shape))
    row_slab = lambda tm, cols: pl.BlockSpec((tm, cols), lambda i: (i, 0))
    f8 = jnp.float8_e4m3fn

    sup23, adjq = _row_slab_call(
        _agg1_kernel, n_p, _TM1,
        (jax.ShapeDtypeStruct((n_p, f23), f8),
         jax.ShapeDtypeStruct((n_p, n_p), f8)),
        (row_slab(_TM1, f23), row_slab(_TM1, n_p)),
        (adj, x, w1, b1, w23),
        [row_slab(_TM1, n_p), whole((n_p, nfeat)), whole((nfeat, nhid)),
         whole((1, nhid)), whole((nhid, f23))],
    )
    slab = _row_slab_call(
        functools.partial(_agg2_kernel, c1=c1, c2=c2),
        n_p, _TM2,
        jax.ShapeDtypeStruct((n_p, f23), jnp.float32), row_slab(_TM2, f23),
        (adjq, sup23, b23),
        [row_slab(_TM2, n_p), whole((n_p, f23)), whole((1, f23))],
    )

    return slab[:n, :c1], slab[:n, c1:c1 + c2], slab[:n, c1 + c2]


# speedup vs baseline: 1.1161x; 1.0092x over previous
import functools

import jax
import jax.numpy as jnp
from jax import lax
from jax.experimental import pallas as pl
from jax.experimental.pallas import tpu as pltpu

_LANE = 128
_TM1 = 512
_TM2 = 2048
_VMEM = 64 * 1024 * 1024
_QSCALE = 128.0


def _ceil_to(v, m):
    return ((v + m - 1) // m) * m


def _agg1_kernel(adj_ref, x_ref, w1_ref, b1_ref, w23_ref, o_ref, adjq_ref):
    a = adj_ref[...]
    adjq_ref[...] = (a * _QSCALE).astype(adjq_ref.dtype)
    t = jnp.dot(
        a.astype(jnp.bfloat16), x_ref[...].astype(jnp.bfloat16),
        preferred_element_type=jnp.float32,
    )
    h = jnp.dot(t, w1_ref[...], preferred_element_type=jnp.float32)
    h = jnp.maximum(h + b1_ref[...], 0.0).astype(jnp.bfloat16)
    o_ref[...] = jnp.dot(
        h, w23_ref[...], preferred_element_type=jnp.float32
    ).astype(o_ref.dtype)


def _agg2_kernel(adjq_ref, sup_ref, b23_ref, o_ref, *, c1, c2):
    z = jnp.dot(adjq_ref[...], sup_ref[...], preferred_element_type=jnp.float32)
    z = z * (1.0 / _QSCALE) + b23_ref[...]

    lane = lax.broadcasted_iota(jnp.int32, z.shape, 1)
    minus_inf = jnp.float32(-jnp.inf)

    def group_stats(mask):
        zg = jnp.where(mask, z, minus_inf)
        m = jnp.max(zg, axis=-1, keepdims=True)
        e = jnp.exp(zg - m)
        return m, e, jnp.sum(e, axis=-1, keepdims=True)

    mask1 = lane < c1
    mask2 = (lane >= c1) & (lane < c1 + c2)
    m1, e1, s1 = group_stats(mask1)
    m2, _, s2 = group_stats(mask2)
    prob_last = (
        jnp.sum(jnp.where(lane == c1 - 1, e1, 0.0), axis=-1, keepdims=True)
        / s1
    )
    out = jnp.where(
        mask1,
        z - m1 - jnp.log(s1),
        jnp.where(mask2, z - m2 - jnp.log(s2), prob_last),
    )
    o_ref[...] = out


def _row_slab_call(body, n_p, tm, out_shapes, out_specs, operands,
                   operand_specs):
    return pl.pallas_call(
        body,
        out_shape=out_shapes,
        grid=(n_p // tm,),
        in_specs=operand_specs,
        out_specs=out_specs,
        compiler_params=pltpu.CompilerParams(
            dimension_semantics=("parallel",),
            vmem_limit_bytes=_VMEM,
        ),
    )(*operands)


def kernel(gc1_w, gc1_b, gc2_w, gc2_b, gc3_w, gc3_b, x, adj):
    n, nfeat = x.shape
    nhid = gc1_w.shape[1]
    c1 = gc2_w.shape[1]
    c2 = gc3_w.shape[1]
    f23 = _ceil_to(c1 + c2 + 1, _LANE)

    n_p = _ceil_to(n, max(_TM1, _TM2))
    if n_p != n:
        x = jnp.pad(x, ((0, n_p - n), (0, 0)))
        adj = jnp.pad(adj, ((0, n_p - n), (0, n_p - n)))

    w1 = gc1_w
    b1 = gc1_b.reshape(1, nhid)
    w23 = jnp.pad(
        jnp.concatenate([gc2_w, gc3_w], axis=1),
        ((0, 0), (0, f23 - c1 - c2)),
    ).astype(jnp.bfloat16)
    b23 = jnp.pad(
        jnp.concatenate([gc2_b, gc3_b]), (0, f23 - c1 - c2)
    ).reshape(1, f23)

    whole = lambda shape: pl.BlockSpec(shape, lambda i: (0,) * len(shape))
    row_slab = lambda tm, cols: pl.BlockSpec((tm, cols), lambda i: (i, 0))
    f8 = jnp.float8_e4m3fn

    sup23, adjq = _row_slab_call(
        _agg1_kernel, n_p, _TM1,
        (jax.ShapeDtypeStruct((n_p, f23), f8),
         jax.ShapeDtypeStruct((n_p, n_p), f8)),
        (row_slab(_TM1, f23), row_slab(_TM1, n_p)),
        (adj, x, w1, b1, w23),
        [row_slab(_TM1, n_p), whole((n_p, nfeat)), whole((nfeat, nhid)),
         whole((1, nhid)), whole((nhid, f23))],
    )
    slab = _row_slab_call(
        functools.partial(_agg2_kernel, c1=c1, c2=c2),
        n_p, _TM2,
        jax.ShapeDtypeStruct((n_p, f23), jnp.float32), row_slab(_TM2, f23),
        (adjq, sup23, b23),
        [row_slab(_TM2, n_p), whole((n_p, f23)), whole((1, f23))],
    )

    return slab[:n, :c1], slab[:n, c1:c1 + c2], slab[:n, c1 + c2]


# speedup vs baseline: 1.1300x; 1.0125x over previous
import functools

import jax
import jax.numpy as jnp
from jax import lax
from jax.experimental import pallas as pl
from jax.experimental.pallas import tpu as pltpu

_LANE = 128
_TM1 = 512
_TM2 = 1024
_VMEM = 64 * 1024 * 1024
_QSCALE = 128.0


def _ceil_to(v, m):
    return ((v + m - 1) // m) * m


def _agg1_kernel(adj_ref, x_ref, w1_ref, b1_ref, w23_ref, o_ref, adjq_ref):
    a = adj_ref[...]
    adjq_ref[...] = (a * _QSCALE).astype(adjq_ref.dtype)
    t = jnp.dot(
        a.astype(jnp.bfloat16), x_ref[...].astype(jnp.bfloat16),
        preferred_element_type=jnp.float32,
    )
    h = jnp.dot(t, w1_ref[...], preferred_element_type=jnp.float32)
    h = jnp.maximum(h + b1_ref[...], 0.0).astype(jnp.bfloat16)
    o_ref[...] = jnp.dot(
        h, w23_ref[...], preferred_element_type=jnp.float32
    ).astype(o_ref.dtype)


def _agg2_kernel(adjq_ref, sup_ref, b23_ref, o_ref, *, c1, c2):
    z = jnp.dot(adjq_ref[...], sup_ref[...], preferred_element_type=jnp.float32)
    z = z * (1.0 / _QSCALE) + b23_ref[...]

    lane = lax.broadcasted_iota(jnp.int32, z.shape, 1)
    minus_inf = jnp.float32(-jnp.inf)

    def group_stats(mask):
        zg = jnp.where(mask, z, minus_inf)
        m = jnp.max(zg, axis=-1, keepdims=True)
        e = jnp.exp(zg - m)
        return m, e, jnp.sum(e, axis=-1, keepdims=True)

    mask1 = lane < c1
    mask2 = (lane >= c1) & (lane < c1 + c2)
    m1, e1, s1 = group_stats(mask1)
    m2, _, s2 = group_stats(mask2)
    prob_last = (
        jnp.sum(jnp.where(lane == c1 - 1, e1, 0.0), axis=-1, keepdims=True)
        / s1
    )
    out = jnp.where(
        mask1,
        z - m1 - jnp.log(s1),
        jnp.where(mask2, z - m2 - jnp.log(s2), prob_last),
    )
    o_ref[...] = out


def _row_slab_call(body, n_p, tm, out_shapes, out_specs, operands,
                   operand_specs):
    return pl.pallas_call(
        body,
        out_shape=out_shapes,
        grid=(n_p // tm,),
        in_specs=operand_specs,
        out_specs=out_specs,
        compiler_params=pltpu.CompilerParams(
            dimension_semantics=("parallel",),
            vmem_limit_bytes=_VMEM,
        ),
    )(*operands)


def kernel(gc1_w, gc1_b, gc2_w, gc2_b, gc3_w, gc3_b, x, adj):
    n, nfeat = x.shape
    nhid = gc1_w.shape[1]
    c1 = gc2_w.shape[1]
    c2 = gc3_w.shape[1]
    f23 = _ceil_to(c1 + c2 + 1, _LANE)

    n_p = _ceil_to(n, max(_TM1, _TM2))
    if n_p != n:
        x = jnp.pad(x, ((0, n_p - n), (0, 0)))
        adj = jnp.pad(adj, ((0, n_p - n), (0, n_p - n)))

    w1 = gc1_w
    b1 = gc1_b.reshape(1, nhid)
    w23 = jnp.pad(
        jnp.concatenate([gc2_w, gc3_w], axis=1),
        ((0, 0), (0, f23 - c1 - c2)),
    ).astype(jnp.bfloat16)
    b23 = jnp.pad(
        jnp.concatenate([gc2_b, gc3_b]), (0, f23 - c1 - c2)
    ).reshape(1, f23)

    whole = lambda shape: pl.BlockSpec(shape, lambda i: (0,) * len(shape))
    row_slab = lambda tm, cols: pl.BlockSpec((tm, cols), lambda i: (i, 0))
    f8 = jnp.float8_e4m3fn

    sup23, adjq = _row_slab_call(
        _agg1_kernel, n_p, _TM1,
        (jax.ShapeDtypeStruct((n_p, f23), f8),
         jax.ShapeDtypeStruct((n_p, n_p), f8)),
        (row_slab(_TM1, f23), row_slab(_TM1, n_p)),
        (adj, x, w1, b1, w23),
        [row_slab(_TM1, n_p), whole((n_p, nfeat)), whole((nfeat, nhid)),
         whole((1, nhid)), whole((nhid, f23))],
    )
    slab = _row_slab_call(
        functools.partial(_agg2_kernel, c1=c1, c2=c2),
        n_p, _TM2,
        jax.ShapeDtypeStruct((n_p, f23), jnp.float32), row_slab(_TM2, f23),
        (adjq, sup23, b23),
        [row_slab(_TM2, n_p), whole((n_p, f23)), whole((1, f23))],
    )

    return slab[:n, :c1], slab[:n, c1:c1 + c2], slab[:n, c1 + c2]
